# baseline (speedup 1.0000x reference)
import functools

import numpy as np
import jax
import jax.numpy as jnp
from jax import lax
from jax.experimental import pallas as pl
from jax.experimental.pallas import tpu as pltpu
from jax.experimental.pallas import tpu_sc as plsc

B = 16384
E = 64
NWF = 18
NTF = 18
NDF = 12
HID = 100
NCLS = 79

NTOT = B * NWF
NW = 32
PER_W = NTOT // NW
CH = 128
SUB = 2
GROUP = CH * SUB
NG = PER_W // GROUP
EP = 128

NFEAT = NTF + NDF
P_ROWS = NFEAT * 48

BT = 1024

_e = np.zeros((NFEAT, P_ROWS), dtype=np.float32)
for _r in range(NFEAT):
    _e[_r, 48 * _r:48 * (_r + 1)] = 1.0
_EXP = _e.astype(jnp.bfloat16)

_MOD = np.tile(np.arange(48, dtype=np.float32), NFEAT)[None, :]


V = 1000003
TW = 13312
TBLK = -(-V // TW)


def _prep_table(wt_t):
    def body(in_ref, out_ref):
        out_ref[:, :E] = jnp.transpose(in_ref[...], (1, 0))

    return pl.pallas_call(
        body,
        grid=(TBLK,),
        in_specs=[pl.BlockSpec((E, TW), lambda j: (0, j))],
        out_specs=pl.BlockSpec((TW, EP), lambda j: (j, 0)),
        out_shape=jax.ShapeDtypeStruct((V, EP), jnp.float32),
    )(wt_t)


def _sc_word_gather(table, idx3, ntot):
    per_w = ntot // NW
    ng = per_w // GROUP
    mesh = plsc.VectorSubcoreMesh(core_axis_name="c", subcore_axis_name="s")

    @functools.partial(
        pl.kernel,
        mesh=mesh,
        compiler_params=pltpu.CompilerParams(use_tc_tiling_on_sc=False),
        out_type=jax.ShapeDtypeStruct((ntot, EP), jnp.float32),
        scratch_types=[
            pltpu.VMEM((per_w // CH, CH), jnp.int32),
            pltpu.VMEM((GROUP, EP), jnp.float32),
            pltpu.VMEM((GROUP, EP), jnp.float32),
            pltpu.SemaphoreType.DMA,
            pltpu.SemaphoreType.DMA,
        ],
    )
    def k(table_h, idx_h, out_h, idx_v, rows_a, rows_b, sem_a, sem_b):
        wid = lax.axis_index("s") * 2 + lax.axis_index("c")
        pltpu.sync_copy(idx_h.at[wid], idx_v)
        base = wid * per_w
        bufs = (rows_a, rows_b)
        sems = (sem_a, sem_b)

        def issue(g, buf, sem):
            for s in range(SUB):
                pltpu.async_copy(
                    table_h.at[idx_v.at[g * SUB + s]],
                    buf.at[pl.ds(s * CH, CH)],
                    sem,
                )

        def drain(buf, sem):
            pltpu.make_async_copy(table_h.at[pl.ds(0, GROUP)], buf,
                                  sem).wait()

        issue(0, rows_a, sem_a)

        def body(h, carry):
            g = 2 * h
            issue(g + 1, rows_b, sem_b)
            drain(rows_a, sem_a)
            pltpu.sync_copy(
                rows_a.at[:, pl.ds(0, E)],
                out_h.at[pl.ds(base + g * GROUP, GROUP), pl.ds(0, E)])

            @pl.when(g + 2 < ng)
            def _():
                issue(g + 2, rows_a, sem_a)

            drain(rows_b, sem_b)
            pltpu.sync_copy(
                rows_b.at[:, pl.ds(0, E)],
                out_h.at[pl.ds(base + (g + 1) * GROUP, GROUP), pl.ds(0, E)])
            return carry

        lax.fori_loop(0, ng // 2, body, 0)

    return k(table, idx3)


def _project_small_tables(tag_table, dep_table, w1_td):
    def body(tag_ref, dep_ref, w_ref, out_ref):
        for f in range(NTF):
            out_ref[48 * f:48 * (f + 1), :] = jnp.dot(
                tag_ref[...], w_ref[E * f:E * (f + 1), :],
                preferred_element_type=jnp.float32).astype(jnp.bfloat16)
        for f in range(NDF):
            base = 48 * (NTF + f)
            out_ref[base:base + 40, :] = jnp.dot(
                dep_ref[...], w_ref[NTF * E + E * f:NTF * E + E * (f + 1), :],
                preferred_element_type=jnp.float32).astype(jnp.bfloat16)
            out_ref[base + 40:base + 48, :] = jnp.zeros(
                (8, HID), dtype=jnp.bfloat16)

    return pl.pallas_call(
        body,
        out_shape=jax.ShapeDtypeStruct((P_ROWS, HID), jnp.bfloat16),
    )(tag_table, dep_table, w1_td)


def _mlp(yv, v_bf, w1w3, p_tbl, exp_mat, mod_row, b1, w2, b2, nb):
    def body(y_ref, v_ref, w_ref, p_ref, e_ref, m_ref, b1_ref, w2_ref,
             b2_ref, out_ref):
        xv = jnp.dot(v_ref[...], e_ref[...],
                     preferred_element_type=jnp.float32)
        oh = (xv == m_ref[...]).astype(jnp.bfloat16)
        acc = jnp.dot(oh, p_ref[...], preferred_element_type=jnp.float32)
        for f in range(NWF):
            acc = acc + jnp.dot(y_ref[f][:, :E].astype(jnp.bfloat16),
                                w_ref[f],
                                preferred_element_type=jnp.float32)
        acc = acc + b1_ref[...]
        h = jnp.maximum(acc, 0.0).astype(jnp.bfloat16)
        out_ref[...] = (jnp.dot(h, w2_ref[...],
                                preferred_element_type=jnp.float32)
                        + b2_ref[...])

    grid = nb // BT
    return pl.pallas_call(
        body,
        grid=(grid,),
        in_specs=[
            pl.BlockSpec((NWF, BT, EP), lambda i: (0, i, 0)),
            pl.BlockSpec((BT, NFEAT), lambda i: (i, 0)),
            pl.BlockSpec((NWF, E, HID), lambda i: (0, 0, 0)),
            pl.BlockSpec((P_ROWS, HID), lambda i: (0, 0)),
            pl.BlockSpec((NFEAT, P_ROWS), lambda i: (0, 0)),
            pl.BlockSpec((1, P_ROWS), lambda i: (0, 0)),
            pl.BlockSpec((1, HID), lambda i: (0, 0)),
            pl.BlockSpec((HID, NCLS), lambda i: (0, 0)),
            pl.BlockSpec((1, NCLS), lambda i: (0, 0)),
        ],
        out_specs=pl.BlockSpec((BT, NCLS), lambda i: (i, 0)),
        out_shape=jax.ShapeDtypeStruct((nb, NCLS), jnp.float32),
    )(yv, v_bf, w1w3, p_tbl, exp_mat, mod_row, b1, w2, b2)


def kernel(X_word, X_tag, X_deprel, word_table, tag_table, deprel_table,
           W1, b1, W2, b2):
    half = B // 2
    nh = half * NWF
    xw32 = X_word.astype(jnp.int32)
    idx3a = xw32[:half].T.reshape(NW, nh // NW // CH, CH)
    idx3b = xw32[half:].T.reshape(NW, nh // NW // CH, CH)
    wt_pad = _prep_table(word_table.T)
    ya = _sc_word_gather(wt_pad, idx3a, nh).reshape(NWF, half, EP)
    yb = _sc_word_gather(wt_pad, idx3b, nh).reshape(NWF, half, EP)

    p_tbl = _project_small_tables(tag_table, deprel_table, W1[NWF * E:, :])

    v_bf = jnp.concatenate(
        [X_tag.astype(jnp.int32), X_deprel.astype(jnp.int32)],
        axis=1).astype(jnp.bfloat16)
    w1w3 = W1[:NWF * E, :].reshape(NWF, E, HID).astype(jnp.bfloat16)
    exp_m = jnp.asarray(_EXP)
    mod_r = jnp.asarray(_MOD)
    b1r = b1.reshape(1, HID)
    w2b = W2.astype(jnp.bfloat16)
    b2r = b2.reshape(1, NCLS)

    la = _mlp(ya, v_bf[:half], w1w3, p_tbl, exp_m, mod_r, b1r, w2b, b2r,
              half)
    lb = _mlp(yb, v_bf[half:], w1w3, p_tbl, exp_m, mod_r, b1r, w2b, b2r,
              half)
    return jnp.concatenate([la, lb], axis=0)


# speedup vs baseline: 1.0122x; 1.0122x over previous
import functools

import numpy as np
import jax
import jax.numpy as jnp
from jax import lax
from jax.experimental import pallas as pl
from jax.experimental.pallas import tpu as pltpu
from jax.experimental.pallas import tpu_sc as plsc

B = 16384
E = 64
NWF = 18
NTF = 18
NDF = 12
HID = 100
NCLS = 79

NTOT = B * NWF
NW = 32
PER_W = NTOT // NW
CH = 128
SUB = 2
GROUP = CH * SUB
NG = PER_W // GROUP
EP = 128

NFEAT = NTF + NDF
P_ROWS = NFEAT * 48

BT = 1024

_e = np.zeros((NFEAT, P_ROWS), dtype=np.float32)
for _r in range(NFEAT):
    _e[_r, 48 * _r:48 * (_r + 1)] = 1.0
_EXP = _e.astype(jnp.bfloat16)

_MOD = np.tile(np.arange(48, dtype=np.float32), NFEAT)[None, :]


V = 1000003
TW = 13312
TBLK = -(-V // TW)


def _prep_table(wt_t):
    def body(in_ref, out_ref):
        out_ref[:, :E] = jnp.transpose(in_ref[...], (1, 0))

    return pl.pallas_call(
        body,
        grid=(TBLK,),
        in_specs=[pl.BlockSpec((E, TW), lambda j: (0, j))],
        out_specs=pl.BlockSpec((TW, EP), lambda j: (j, 0)),
        out_shape=jax.ShapeDtypeStruct((V, EP), jnp.float32),
    )(wt_t)


def _sc_word_gather(table, idx3):
    mesh = plsc.VectorSubcoreMesh(core_axis_name="c", subcore_axis_name="s")

    @functools.partial(
        pl.kernel,
        mesh=mesh,
        compiler_params=pltpu.CompilerParams(use_tc_tiling_on_sc=False),
        out_type=jax.ShapeDtypeStruct((NTOT, EP), jnp.float32),
        scratch_types=[
            pltpu.VMEM((PER_W // CH, CH), jnp.int32),
            pltpu.VMEM((GROUP, EP), jnp.float32),
            pltpu.VMEM((GROUP, EP), jnp.float32),
            pltpu.SemaphoreType.DMA,
            pltpu.SemaphoreType.DMA,
        ],
    )
    def k(table_h, idx_h, out_h, idx_v, rows_a, rows_b, sem_a, sem_b):
        wid = lax.axis_index("s") * 2 + lax.axis_index("c")
        pltpu.sync_copy(idx_h.at[wid], idx_v)
        base = wid * PER_W
        bufs = (rows_a, rows_b)
        sems = (sem_a, sem_b)

        def issue(g, buf, sem):
            for s in range(SUB):
                pltpu.async_copy(
                    table_h.at[idx_v.at[g * SUB + s]],
                    buf.at[pl.ds(s * CH, CH)],
                    sem,
                )

        def drain(buf, sem):
            pltpu.make_async_copy(table_h.at[pl.ds(0, GROUP)], buf,
                                  sem).wait()

        issue(0, rows_a, sem_a)

        def body(h, carry):
            g = 2 * h
            issue(g + 1, rows_b, sem_b)
            drain(rows_a, sem_a)
            pltpu.sync_copy(
                rows_a.at[:, pl.ds(0, E)],
                out_h.at[pl.ds(base + g * GROUP, GROUP), pl.ds(0, E)])

            @pl.when(g + 2 < NG)
            def _():
                issue(g + 2, rows_a, sem_a)

            drain(rows_b, sem_b)
            pltpu.sync_copy(
                rows_b.at[:, pl.ds(0, E)],
                out_h.at[pl.ds(base + (g + 1) * GROUP, GROUP), pl.ds(0, E)])
            return carry

        lax.fori_loop(0, NG // 2, body, 0)

    return k(table, idx3)


def _project_small_tables(tag_table, dep_table, w1_td):
    def body(tag_ref, dep_ref, w_ref, out_ref):
        for f in range(NTF):
            out_ref[48 * f:48 * (f + 1), :] = jnp.dot(
                tag_ref[...], w_ref[E * f:E * (f + 1), :],
                preferred_element_type=jnp.float32).astype(jnp.bfloat16)
        for f in range(NDF):
            base = 48 * (NTF + f)
            out_ref[base:base + 40, :] = jnp.dot(
                dep_ref[...], w_ref[NTF * E + E * f:NTF * E + E * (f + 1), :],
                preferred_element_type=jnp.float32).astype(jnp.bfloat16)
            out_ref[base + 40:base + 48, :] = jnp.zeros(
                (8, HID), dtype=jnp.bfloat16)

    return pl.pallas_call(
        body,
        out_shape=jax.ShapeDtypeStruct((P_ROWS, HID), jnp.bfloat16),
    )(tag_table, dep_table, w1_td)


def _mlp(yv, v_bf, w1w3, p_tbl, exp_mat, mod_row, b1, w2, b2):
    def body(y_ref, v_ref, w_ref, p_ref, e_ref, m_ref, b1_ref, w2_ref,
             b2_ref, out_ref):
        xv = jnp.dot(v_ref[...], e_ref[...],
                     preferred_element_type=jnp.float32)
        oh = (xv == m_ref[...]).astype(jnp.bfloat16)
        acc = jnp.dot(oh, p_ref[...], preferred_element_type=jnp.float32)
        for f in range(NWF):
            acc = acc + jnp.dot(y_ref[f][:, :E].astype(jnp.bfloat16),
                                w_ref[f],
                                preferred_element_type=jnp.float32)
        acc = acc + b1_ref[...]
        h = jnp.maximum(acc, 0.0).astype(jnp.bfloat16)
        out_ref[...] = (jnp.dot(h, w2_ref[...],
                                preferred_element_type=jnp.float32)
                        + b2_ref[...])

    grid = B // BT
    return pl.pallas_call(
        body,
        grid=(grid,),
        in_specs=[
            pl.BlockSpec((NWF, BT, EP), lambda i: (0, i, 0)),
            pl.BlockSpec((BT, NFEAT), lambda i: (i, 0)),
            pl.BlockSpec((NWF, E, HID), lambda i: (0, 0, 0)),
            pl.BlockSpec((P_ROWS, HID), lambda i: (0, 0)),
            pl.BlockSpec((NFEAT, P_ROWS), lambda i: (0, 0)),
            pl.BlockSpec((1, P_ROWS), lambda i: (0, 0)),
            pl.BlockSpec((1, HID), lambda i: (0, 0)),
            pl.BlockSpec((HID, NCLS), lambda i: (0, 0)),
            pl.BlockSpec((1, NCLS), lambda i: (0, 0)),
        ],
        out_specs=pl.BlockSpec((BT, NCLS), lambda i: (i, 0)),
        out_shape=jax.ShapeDtypeStruct((B, NCLS), jnp.float32),
    )(yv, v_bf, w1w3, p_tbl, exp_mat, mod_row, b1, w2, b2)


def kernel(X_word, X_tag, X_deprel, word_table, tag_table, deprel_table,
           W1, b1, W2, b2):
    idx3 = X_word.astype(jnp.int32).T.reshape(NW, PER_W // CH, CH)
    wt_pad = _prep_table(word_table.T)
    yv = _sc_word_gather(wt_pad, idx3).reshape(NWF, B, EP)

    p_tbl = _project_small_tables(tag_table, deprel_table, W1[NWF * E:, :])

    v_bf = jnp.concatenate(
        [X_tag.astype(jnp.int32), X_deprel.astype(jnp.int32)],
        axis=1).astype(jnp.bfloat16)
    w1w3 = W1[:NWF * E, :].reshape(NWF, E, HID).astype(jnp.bfloat16)

    return _mlp(yv, v_bf, w1w3, p_tbl, jnp.asarray(_EXP),
                jnp.asarray(_MOD), b1.reshape(1, HID),
                W2.astype(jnp.bfloat16), b2.reshape(1, NCLS))
